# baseline (speedup 1.0000x reference)
import functools

import jax
import jax.numpy as jnp
from jax.experimental import pallas as pl
from jax.experimental.pallas import tpu as pltpu

K = 8
LOAD_BALANCE_WEIGHT = 0.01


def _router_kernel(x_ref, wgt_ref, rw_ref, idx_ref, probs_ref, tpe_ref,
                   sprob_ref, loss_ref, *, num_tokens, nsteps):
    step = pl.program_id(0)
    E = wgt_ref.shape[1]

    logits = jnp.dot(x_ref[...], wgt_ref[...],
                     preferred_element_type=jnp.float32)
    m = jnp.max(logits, axis=-1, keepdims=True)
    e = jnp.exp(logits - m)
    probs = e / jnp.sum(e, axis=-1, keepdims=True)
    probs_ref[...] = probs

    tb = logits.shape[0]
    iota = jax.lax.broadcasted_iota(jnp.int32, (tb, E), 1)
    work = probs
    selected = jnp.zeros((tb, E), dtype=jnp.float32)
    vals = []
    idxs = []
    for _ in range(K):
        cur = jnp.max(work, axis=-1, keepdims=True)
        cand = jnp.where(work == cur, iota, E)
        ci = jnp.min(cand, axis=-1, keepdims=True)
        vals.append(cur)
        idxs.append(ci)
        hit = (iota == ci).astype(jnp.float32)
        selected = selected + hit
        work = jnp.where(iota == ci, -jnp.inf, work)

    top_vals = jnp.concatenate(vals, axis=1)
    top_idx = jnp.concatenate(idxs, axis=1)
    rw_ref[...] = top_vals / jnp.sum(top_vals, axis=-1, keepdims=True)
    idx_ref[...] = top_idx

    blk_counts = jnp.sum(selected, axis=0, keepdims=True)
    blk_psum = jnp.sum(probs, axis=0, keepdims=True)

    @pl.when(step == 0)
    def _init():
        tpe_ref[...] = blk_counts
        sprob_ref[...] = blk_psum

    @pl.when(step != 0)
    def _acc():
        tpe_ref[...] += blk_counts
        sprob_ref[...] += blk_psum

    @pl.when(step == nsteps - 1)
    def _finish():
        frac = tpe_ref[...] / num_tokens
        mean_prob = sprob_ref[...] / num_tokens
        loss_ref[...] = (LOAD_BALANCE_WEIGHT * E
                         * jnp.sum(frac * mean_prob, keepdims=True))


@jax.jit
def kernel(x, Wg, Wn):
    B, S, D = x.shape
    E = Wg.shape[0]
    T = B * S
    TB = 256
    nsteps = T // TB

    xt = x.reshape(T, D)
    wgt = Wg.T

    grid = (nsteps,)
    out_shapes = (
        jax.ShapeDtypeStruct((T, K), jnp.float32),
        jax.ShapeDtypeStruct((T, K), jnp.int32),
        jax.ShapeDtypeStruct((T, E), jnp.float32),
        jax.ShapeDtypeStruct((1, E), jnp.float32),
        jax.ShapeDtypeStruct((1, E), jnp.float32),
        jax.ShapeDtypeStruct((1, 1), jnp.float32),
    )
    in_specs = [
        pl.BlockSpec((TB, D), lambda i: (i, 0)),
        pl.BlockSpec((D, E), lambda i: (0, 0)),
    ]
    out_specs = (
        pl.BlockSpec((TB, K), lambda i: (i, 0)),
        pl.BlockSpec((TB, K), lambda i: (i, 0)),
        pl.BlockSpec((TB, E), lambda i: (i, 0)),
        pl.BlockSpec((1, E), lambda i: (0, 0)),
        pl.BlockSpec((1, E), lambda i: (0, 0)),
        pl.BlockSpec((1, 1), lambda i: (0, 0)),
    )
    rw, idx, probs, _tpe, _sprob, loss = pl.pallas_call(
        functools.partial(_router_kernel, num_tokens=float(T), nsteps=nsteps),
        grid=grid,
        in_specs=in_specs,
        out_specs=out_specs,
        out_shape=out_shapes,
        compiler_params=pltpu.CompilerParams(
            dimension_semantics=("arbitrary",),
        ),
    )(xt, wgt)

    return (rw.reshape(B, S, K),
            idx.reshape(B, S, K),
            loss.reshape(()),
            probs.reshape(B, S, E))


# speedup vs baseline: 1.6695x; 1.6695x over previous
import functools

import jax
import jax.numpy as jnp
from jax.experimental import pallas as pl
from jax.experimental.pallas import tpu as pltpu

K = 8
LOAD_BALANCE_WEIGHT = 0.01


def _router_kernel(x_ref, wgt_ref, rw_ref, idx_ref, probs_ref, tpe_ref,
                   sprob_ref, loss_ref, *, num_tokens, nsteps):
    step = pl.program_id(0)
    E = wgt_ref.shape[1]

    logits = jnp.dot(x_ref[...], wgt_ref[...],
                     preferred_element_type=jnp.float32)
    m = jnp.max(logits, axis=-1, keepdims=True)
    e = jnp.exp(logits - m)
    probs = e / jnp.sum(e, axis=-1, keepdims=True)
    probs_ref[...] = probs

    tb = logits.shape[0]
    work = probs.T
    riota = jax.lax.broadcasted_iota(jnp.int32, (E, tb), 0)
    selected = jnp.zeros((E, tb), dtype=jnp.float32)
    vals = []
    idxs = []
    for _ in range(K):
        cur = jnp.max(work, axis=0, keepdims=True)
        cand = jnp.where(work == cur, riota, E)
        ci = jnp.min(cand, axis=0, keepdims=True)
        vals.append(cur)
        idxs.append(ci)
        hit = (riota == ci).astype(jnp.float32)
        selected = selected + hit
        work = jnp.where(riota == ci, -jnp.inf, work)

    top_vals = jnp.concatenate(vals, axis=0)
    top_idx = jnp.concatenate(idxs, axis=0)
    norm = top_vals / jnp.sum(top_vals, axis=0, keepdims=True)
    rw_ref[...] = norm.T
    idx_ref[...] = top_idx.T

    blk_counts = jnp.sum(selected, axis=1, keepdims=True).T
    blk_psum = jnp.sum(probs, axis=0, keepdims=True)

    @pl.when(step == 0)
    def _init():
        tpe_ref[...] = blk_counts
        sprob_ref[...] = blk_psum

    @pl.when(step != 0)
    def _acc():
        tpe_ref[...] += blk_counts
        sprob_ref[...] += blk_psum

    @pl.when(step == nsteps - 1)
    def _finish():
        frac = tpe_ref[...] / num_tokens
        mean_prob = sprob_ref[...] / num_tokens
        loss_ref[...] = (LOAD_BALANCE_WEIGHT * E
                         * jnp.sum(frac * mean_prob, keepdims=True))


@jax.jit
def kernel(x, Wg, Wn):
    B, S, D = x.shape
    E = Wg.shape[0]
    T = B * S
    TB = 512
    nsteps = T // TB

    xt = x.reshape(T, D)
    wgt = Wg.T

    grid = (nsteps,)
    out_shapes = (
        jax.ShapeDtypeStruct((T, K), jnp.float32),
        jax.ShapeDtypeStruct((T, K), jnp.int32),
        jax.ShapeDtypeStruct((T, E), jnp.float32),
        jax.ShapeDtypeStruct((1, E), jnp.float32),
        jax.ShapeDtypeStruct((1, E), jnp.float32),
        jax.ShapeDtypeStruct((1, 1), jnp.float32),
    )
    in_specs = [
        pl.BlockSpec((TB, D), lambda i: (i, 0)),
        pl.BlockSpec((D, E), lambda i: (0, 0)),
    ]
    out_specs = (
        pl.BlockSpec((TB, K), lambda i: (i, 0)),
        pl.BlockSpec((TB, K), lambda i: (i, 0)),
        pl.BlockSpec((TB, E), lambda i: (i, 0)),
        pl.BlockSpec((1, E), lambda i: (0, 0)),
        pl.BlockSpec((1, E), lambda i: (0, 0)),
        pl.BlockSpec((1, 1), lambda i: (0, 0)),
    )
    rw, idx, probs, _tpe, _sprob, loss = pl.pallas_call(
        functools.partial(_router_kernel, num_tokens=float(T), nsteps=nsteps),
        grid=grid,
        in_specs=in_specs,
        out_specs=out_specs,
        out_shape=out_shapes,
        compiler_params=pltpu.CompilerParams(
            dimension_semantics=("arbitrary",),
        ),
    )(xt, wgt)

    return (rw.reshape(B, S, K),
            idx.reshape(B, S, K),
            loss.reshape(()),
            probs.reshape(B, S, E))


# speedup vs baseline: 1.8167x; 1.0882x over previous
import functools

import jax
import jax.numpy as jnp
from jax.experimental import pallas as pl
from jax.experimental.pallas import tpu as pltpu

K = 8
LOAD_BALANCE_WEIGHT = 0.01


def _router_kernel(x_ref, wgt_ref, rw_ref, idx_ref, probs_ref, tpe_ref,
                   sprob_ref, loss_ref, *, num_tokens, nsteps):
    step = pl.program_id(0)
    E = wgt_ref.shape[1]

    logits = jnp.dot(x_ref[...], wgt_ref[...],
                     preferred_element_type=jnp.float32)
    m = jnp.max(logits, axis=-1, keepdims=True)
    e = jnp.exp(logits - m)
    probs = e / jnp.sum(e, axis=-1, keepdims=True)
    probs_ref[...] = probs

    tb = logits.shape[0]
    work = probs.T
    riota = jax.lax.broadcasted_iota(jnp.int32, (E, tb), 0)
    selected = jnp.zeros((E, tb), dtype=jnp.float32)
    vals = []
    idxs = []
    for _ in range(K):
        cur = jnp.max(work, axis=0, keepdims=True)
        cand = jnp.where(work == cur, riota, E)
        ci = jnp.min(cand, axis=0, keepdims=True)
        vals.append(cur)
        idxs.append(ci)
        hit = (riota == ci).astype(jnp.float32)
        selected = selected + hit
        work = jnp.where(riota == ci, -jnp.inf, work)

    top_vals = jnp.concatenate(vals, axis=0)
    top_idx = jnp.concatenate(idxs, axis=0)
    norm = top_vals / jnp.sum(top_vals, axis=0, keepdims=True)
    rw_ref[...] = norm.T
    idx_ref[...] = top_idx.T

    blk_counts = jnp.sum(selected, axis=1, keepdims=True).T
    blk_psum = jnp.sum(probs, axis=0, keepdims=True)

    @pl.when(step == 0)
    def _init():
        tpe_ref[...] = blk_counts
        sprob_ref[...] = blk_psum

    @pl.when(step != 0)
    def _acc():
        tpe_ref[...] += blk_counts
        sprob_ref[...] += blk_psum

    @pl.when(step == nsteps - 1)
    def _finish():
        frac = tpe_ref[...] / num_tokens
        mean_prob = sprob_ref[...] / num_tokens
        loss_ref[...] = (LOAD_BALANCE_WEIGHT * E
                         * jnp.sum(frac * mean_prob, keepdims=True))


@jax.jit
def kernel(x, Wg, Wn):
    B, S, D = x.shape
    E = Wg.shape[0]
    T = B * S
    TB = 1024
    nsteps = T // TB

    xt = x.reshape(T, D)
    wgt = Wg.T

    grid = (nsteps,)
    out_shapes = (
        jax.ShapeDtypeStruct((T, K), jnp.float32),
        jax.ShapeDtypeStruct((T, K), jnp.int32),
        jax.ShapeDtypeStruct((T, E), jnp.float32),
        jax.ShapeDtypeStruct((1, E), jnp.float32),
        jax.ShapeDtypeStruct((1, E), jnp.float32),
        jax.ShapeDtypeStruct((1, 1), jnp.float32),
    )
    in_specs = [
        pl.BlockSpec((TB, D), lambda i: (i, 0)),
        pl.BlockSpec((D, E), lambda i: (0, 0)),
    ]
    out_specs = (
        pl.BlockSpec((TB, K), lambda i: (i, 0)),
        pl.BlockSpec((TB, K), lambda i: (i, 0)),
        pl.BlockSpec((TB, E), lambda i: (i, 0)),
        pl.BlockSpec((1, E), lambda i: (0, 0)),
        pl.BlockSpec((1, E), lambda i: (0, 0)),
        pl.BlockSpec((1, 1), lambda i: (0, 0)),
    )
    rw, idx, probs, _tpe, _sprob, loss = pl.pallas_call(
        functools.partial(_router_kernel, num_tokens=float(T), nsteps=nsteps),
        grid=grid,
        in_specs=in_specs,
        out_specs=out_specs,
        out_shape=out_shapes,
        compiler_params=pltpu.CompilerParams(
            dimension_semantics=("arbitrary",),
        ),
    )(xt, wgt)

    return (rw.reshape(B, S, K),
            idx.reshape(B, S, K),
            loss.reshape(()),
            probs.reshape(B, S, E))


# speedup vs baseline: 1.8713x; 1.0301x over previous
import functools

import jax
import jax.numpy as jnp
from jax.experimental import pallas as pl
from jax.experimental.pallas import tpu as pltpu

K = 8
LOAD_BALANCE_WEIGHT = 0.01


def _router_kernel(x_ref, wg_ref, rw_ref, idx_ref, probs_ref, tpe_ref,
                   sprob_ref, loss_ref, *, num_tokens, nb, ns):
    b = pl.program_id(0)
    s = pl.program_id(1)
    E = wg_ref.shape[0]

    logits = jax.lax.dot_general(
        x_ref[0], wg_ref[...], (((1,), (1,)), ((), ())),
        preferred_element_type=jnp.float32)
    m = jnp.max(logits, axis=-1, keepdims=True)
    e = jnp.exp(logits - m)
    probs = e / jnp.sum(e, axis=-1, keepdims=True)
    probs_ref[0] = probs

    tb = logits.shape[0]
    work = probs.T
    riota = jax.lax.broadcasted_iota(jnp.int32, (E, tb), 0)
    selected = jnp.zeros((E, tb), dtype=jnp.float32)
    vals = []
    idxs = []
    for _ in range(K):
        cur = jnp.max(work, axis=0, keepdims=True)
        cand = jnp.where(work == cur, riota, E)
        ci = jnp.min(cand, axis=0, keepdims=True)
        vals.append(cur)
        idxs.append(ci)
        hit = (riota == ci).astype(jnp.float32)
        selected = selected + hit
        work = jnp.where(riota == ci, -jnp.inf, work)

    top_vals = jnp.concatenate(vals, axis=0)
    top_idx = jnp.concatenate(idxs, axis=0)
    norm = top_vals / jnp.sum(top_vals, axis=0, keepdims=True)
    rw_ref[0] = norm.T
    idx_ref[0] = top_idx.T

    blk_counts = jnp.sum(selected, axis=1, keepdims=True).T
    blk_psum = jnp.sum(probs, axis=0, keepdims=True)

    first = jnp.logical_and(b == 0, s == 0)
    last = jnp.logical_and(b == nb - 1, s == ns - 1)

    @pl.when(first)
    def _init():
        tpe_ref[...] = blk_counts
        sprob_ref[...] = blk_psum

    @pl.when(jnp.logical_not(first))
    def _acc():
        tpe_ref[...] += blk_counts
        sprob_ref[...] += blk_psum

    @pl.when(last)
    def _finish():
        frac = tpe_ref[...] / num_tokens
        mean_prob = sprob_ref[...] / num_tokens
        loss_ref[...] = (LOAD_BALANCE_WEIGHT * E
                         * jnp.sum(frac * mean_prob, keepdims=True))


@jax.jit
def kernel(x, Wg, Wn):
    B, S, D = x.shape
    E = Wg.shape[0]
    T = B * S
    TB = 1024
    ns = S // TB

    grid = (B, ns)
    out_shapes = (
        jax.ShapeDtypeStruct((B, S, K), jnp.float32),
        jax.ShapeDtypeStruct((B, S, K), jnp.int32),
        jax.ShapeDtypeStruct((B, S, E), jnp.float32),
        jax.ShapeDtypeStruct((1, E), jnp.float32),
        jax.ShapeDtypeStruct((1, E), jnp.float32),
        jax.ShapeDtypeStruct((1, 1), jnp.float32),
    )
    in_specs = [
        pl.BlockSpec((1, TB, D), lambda b, s: (b, s, 0)),
        pl.BlockSpec((E, D), lambda b, s: (0, 0)),
    ]
    out_specs = (
        pl.BlockSpec((1, TB, K), lambda b, s: (b, s, 0)),
        pl.BlockSpec((1, TB, K), lambda b, s: (b, s, 0)),
        pl.BlockSpec((1, TB, E), lambda b, s: (b, s, 0)),
        pl.BlockSpec((1, E), lambda b, s: (0, 0)),
        pl.BlockSpec((1, E), lambda b, s: (0, 0)),
        pl.BlockSpec((1, 1), lambda b, s: (0, 0)),
    )
    rw, idx, probs, _tpe, _sprob, loss = pl.pallas_call(
        functools.partial(_router_kernel, num_tokens=float(T), nb=B, ns=ns),
        grid=grid,
        in_specs=in_specs,
        out_specs=out_specs,
        out_shape=out_shapes,
        compiler_params=pltpu.CompilerParams(
            dimension_semantics=("arbitrary", "arbitrary"),
        ),
    )(x, Wg)

    return (rw, idx, loss.reshape(()), probs)


# speedup vs baseline: 1.8824x; 1.0059x over previous
import functools

import jax
import jax.numpy as jnp
from jax.experimental import pallas as pl
from jax.experimental.pallas import tpu as pltpu

K = 8
LOAD_BALANCE_WEIGHT = 0.01


def _router_kernel(x_ref, wg_ref, rw_ref, idx_ref, probs_ref, tpe_ref,
                   sprob_ref, loss_ref, *, num_tokens, nb, ns):
    b = pl.program_id(0)
    s = pl.program_id(1)
    E = wg_ref.shape[0]

    logits = jax.lax.dot_general(
        x_ref[0], wg_ref[...], (((1,), (1,)), ((), ())),
        preferred_element_type=jnp.float32)
    tb = logits.shape[0]
    lt = logits.T
    m = jnp.max(lt, axis=0, keepdims=True)
    e = jnp.exp(lt - m)
    pt = e / jnp.sum(e, axis=0, keepdims=True)
    probs_ref[0] = pt.T

    riota = jax.lax.broadcasted_iota(jnp.int32, (E, tb), 0).astype(jnp.float32)
    work = pt
    vals = []
    idxs = []
    for _ in range(K):
        cur = jnp.max(work, axis=0, keepdims=True)
        cand = jnp.where(work == cur, riota, float(E))
        ci = jnp.min(cand, axis=0, keepdims=True)
        vals.append(cur)
        idxs.append(ci)
        work = jnp.where(riota == ci, -jnp.inf, work)

    top_vals = jnp.concatenate(vals, axis=0)
    top_idx = jnp.concatenate(idxs, axis=0).astype(jnp.int32)
    norm = top_vals / jnp.sum(top_vals, axis=0, keepdims=True)
    rw_ref[0] = norm.T
    idx_ref[0] = top_idx.T

    sel = (work == -jnp.inf).astype(jnp.float32)
    blk_counts = jnp.sum(sel, axis=1, keepdims=True).T
    blk_psum = jnp.sum(pt, axis=1, keepdims=True).T

    first = jnp.logical_and(b == 0, s == 0)
    last = jnp.logical_and(b == nb - 1, s == ns - 1)

    @pl.when(first)
    def _init():
        tpe_ref[...] = blk_counts
        sprob_ref[...] = blk_psum

    @pl.when(jnp.logical_not(first))
    def _acc():
        tpe_ref[...] += blk_counts
        sprob_ref[...] += blk_psum

    @pl.when(last)
    def _finish():
        frac = tpe_ref[...] / num_tokens
        mean_prob = sprob_ref[...] / num_tokens
        loss_ref[...] = (LOAD_BALANCE_WEIGHT * E
                         * jnp.sum(frac * mean_prob, keepdims=True))


@jax.jit
def kernel(x, Wg, Wn):
    B, S, D = x.shape
    E = Wg.shape[0]
    T = B * S
    TB = 1024
    ns = S // TB

    grid = (B, ns)
    out_shapes = (
        jax.ShapeDtypeStruct((B, S, K), jnp.float32),
        jax.ShapeDtypeStruct((B, S, K), jnp.int32),
        jax.ShapeDtypeStruct((B, S, E), jnp.float32),
        jax.ShapeDtypeStruct((1, E), jnp.float32),
        jax.ShapeDtypeStruct((1, E), jnp.float32),
        jax.ShapeDtypeStruct((1, 1), jnp.float32),
    )
    in_specs = [
        pl.BlockSpec((1, TB, D), lambda b, s: (b, s, 0)),
        pl.BlockSpec((E, D), lambda b, s: (0, 0)),
    ]
    out_specs = (
        pl.BlockSpec((1, TB, K), lambda b, s: (b, s, 0)),
        pl.BlockSpec((1, TB, K), lambda b, s: (b, s, 0)),
        pl.BlockSpec((1, TB, E), lambda b, s: (b, s, 0)),
        pl.BlockSpec((1, E), lambda b, s: (0, 0)),
        pl.BlockSpec((1, E), lambda b, s: (0, 0)),
        pl.BlockSpec((1, 1), lambda b, s: (0, 0)),
    )
    rw, idx, probs, _tpe, _sprob, loss = pl.pallas_call(
        functools.partial(_router_kernel, num_tokens=float(T), nb=B, ns=ns),
        grid=grid,
        in_specs=in_specs,
        out_specs=out_specs,
        out_shape=out_shapes,
        compiler_params=pltpu.CompilerParams(
            dimension_semantics=("arbitrary", "arbitrary"),
        ),
    )(x, Wg)

    return (rw, idx, loss.reshape(()), probs)


# speedup vs baseline: 1.8845x; 1.0011x over previous
import functools

import jax
import jax.numpy as jnp
from jax.experimental import pallas as pl
from jax.experimental.pallas import tpu as pltpu

K = 8
LOAD_BALANCE_WEIGHT = 0.01


def _router_kernel(x_ref, wg_ref, rw_ref, idx_ref, probs_ref, tpe_ref,
                   sprob_ref, loss_ref, *, num_tokens, nb, ns):
    b = pl.program_id(0)
    s = pl.program_id(1)
    E = wg_ref.shape[0]

    logits = jax.lax.dot_general(
        x_ref[0], wg_ref[...], (((1,), (1,)), ((), ())),
        preferred_element_type=jnp.float32)
    tb = logits.shape[0]
    m = jnp.max(logits, axis=-1, keepdims=True)
    e = jnp.exp(logits - m)
    probs = e / jnp.sum(e, axis=-1, keepdims=True)
    probs_ref[0] = probs
    pt = probs.T

    riota = jax.lax.broadcasted_iota(jnp.int32, (E, tb), 0).astype(jnp.float32)
    work = pt
    vals = []
    idxs = []
    for _ in range(K):
        cur = jnp.max(work, axis=0, keepdims=True)
        cand = jnp.where(work == cur, riota, float(E))
        ci = jnp.min(cand, axis=0, keepdims=True)
        vals.append(cur)
        idxs.append(ci)
        work = jnp.where(riota == ci, -jnp.inf, work)

    top_vals = jnp.concatenate(vals, axis=0)
    top_idx = jnp.concatenate(idxs, axis=0).astype(jnp.int32)
    norm = top_vals / jnp.sum(top_vals, axis=0, keepdims=True)
    rw_ref[0] = norm.T
    idx_ref[0] = top_idx.T

    sel = (work == -jnp.inf).astype(jnp.float32)
    blk_counts = jnp.sum(sel, axis=1, keepdims=True).T
    blk_psum = jnp.sum(pt, axis=1, keepdims=True).T

    first = jnp.logical_and(b == 0, s == 0)
    last = jnp.logical_and(b == nb - 1, s == ns - 1)

    @pl.when(first)
    def _init():
        tpe_ref[...] = blk_counts
        sprob_ref[...] = blk_psum

    @pl.when(jnp.logical_not(first))
    def _acc():
        tpe_ref[...] += blk_counts
        sprob_ref[...] += blk_psum

    @pl.when(last)
    def _finish():
        frac = tpe_ref[...] / num_tokens
        mean_prob = sprob_ref[...] / num_tokens
        loss_ref[...] = (LOAD_BALANCE_WEIGHT * E
                         * jnp.sum(frac * mean_prob, keepdims=True))


@jax.jit
def kernel(x, Wg, Wn):
    B, S, D = x.shape
    E = Wg.shape[0]
    T = B * S
    TB = 1024
    ns = S // TB

    grid = (B, ns)
    out_shapes = (
        jax.ShapeDtypeStruct((B, S, K), jnp.float32),
        jax.ShapeDtypeStruct((B, S, K), jnp.int32),
        jax.ShapeDtypeStruct((B, S, E), jnp.float32),
        jax.ShapeDtypeStruct((1, E), jnp.float32),
        jax.ShapeDtypeStruct((1, E), jnp.float32),
        jax.ShapeDtypeStruct((1, 1), jnp.float32),
    )
    in_specs = [
        pl.BlockSpec((1, TB, D), lambda b, s: (b, s, 0)),
        pl.BlockSpec((E, D), lambda b, s: (0, 0)),
    ]
    out_specs = (
        pl.BlockSpec((1, TB, K), lambda b, s: (b, s, 0)),
        pl.BlockSpec((1, TB, K), lambda b, s: (b, s, 0)),
        pl.BlockSpec((1, TB, E), lambda b, s: (b, s, 0)),
        pl.BlockSpec((1, E), lambda b, s: (0, 0)),
        pl.BlockSpec((1, E), lambda b, s: (0, 0)),
        pl.BlockSpec((1, 1), lambda b, s: (0, 0)),
    )
    rw, idx, probs, _tpe, _sprob, loss = pl.pallas_call(
        functools.partial(_router_kernel, num_tokens=float(T), nb=B, ns=ns),
        grid=grid,
        in_specs=in_specs,
        out_specs=out_specs,
        out_shape=out_shapes,
        compiler_params=pltpu.CompilerParams(
            dimension_semantics=("arbitrary", "arbitrary"),
        ),
    )(x, Wg)

    return (rw, idx, loss.reshape(()), probs)
